# baseline (speedup 1.0000x reference)
import functools

import jax
import jax.numpy as jnp
from jax import lax
from jax.experimental import pallas as pl
from jax.experimental.pallas import tpu as pltpu
from jax.experimental.pallas import tpu_sc as plsc

_D = 32
_CH = 1024
_LANES = 16


def _detrans_call(n_emb):
    mesh = plsc.VectorSubcoreMesh(core_axis_name="c", subcore_axis_name="s")
    info = plsc.get_sparse_core_info()
    nc = info.num_cores
    nw = nc * info.num_subcores
    n_slab = (n_emb + 127) // 128
    n_rows_out = (n_emb * _D) // 128
    k_max = (n_slab + nw - 1) // nw

    @functools.partial(
        pl.kernel,
        mesh=mesh,
        out_type=jax.ShapeDtypeStruct((n_rows_out, 128), jnp.float32),
        compiler_params=pltpu.CompilerParams(
            use_tc_tiling_on_sc=True, needs_layout_passes=False),
        scratch_types=[
            pltpu.VMEM((2, _D, 128), jnp.float32),
            pltpu.VMEM((2, _D, 128), jnp.float32),
            pltpu.SemaphoreType.DMA,
            pltpu.SemaphoreType.DMA,
            pltpu.SemaphoreType.DMA,
            pltpu.SemaphoreType.DMA,
        ],
    )
    def detrans(wt_hbm, tail_hbm, ta_hbm, vin, vout, i0, i1, o0, o1):
        wid = lax.axis_index("s") * nc + lax.axis_index("c")
        isem = (i0, i1)
        osem = (o0, o1)
        iota = lax.iota(jnp.int32, _LANES)
        row_half = (iota, iota + _LANES)

        def transpose_slab(slot, n_t):
            for t in range(n_t):
                for g in range(8):
                    rows = row_half[g % 2]
                    cols = jnp.full((_LANES,), 4 * t + g // 2, jnp.int32)
                    val = plsc.load_gather(vin.at[slot], [rows, cols])
                    vout[slot, t, pl.ds(16 * g, 16)] = val

        def fire_in(sl, slot):
            return pltpu.async_copy(
                wt_hbm.at[:, pl.ds(sl * 128, 128)], vin.at[slot], isem[slot])

        def fire_out(sl, slot, n_t):
            return pltpu.async_copy(
                vout.at[slot, pl.ds(0, n_t)],
                ta_hbm.at[pl.ds(sl * _D, n_t)], osem[slot])

        def do_slab(k, slot):
            sl = wid + nw * k

            @pl.when(sl < n_slab - 1)
            def _full():
                fire_in(sl, slot).wait()
                transpose_slab(slot, _D)
                fire_out(sl, slot, _D).wait()

            @pl.when(sl == n_slab - 1)
            def _partial():
                pltpu.async_copy(tail_hbm, vin.at[slot], isem[slot]).wait()
                transpose_slab(slot, _D // 2)
                fire_out(sl, slot, _D // 2).wait()

        def step(m, carry):
            do_slab(2 * m, 0)
            do_slab(2 * m + 1, 1)
            return carry

        lax.fori_loop(0, k_max // 2, step, 0)
        if k_max % 2:
            do_slab(k_max - 1, 0)

    return detrans


def _emb_call(total, n_ch, b_per_w, n_emb):
    mesh = plsc.VectorSubcoreMesh(core_axis_name="c", subcore_axis_name="s")
    info = plsc.get_sparse_core_info()
    nc = info.num_cores

    @functools.partial(
        pl.kernel,
        mesh=mesh,
        out_type=jax.ShapeDtypeStruct((total, _D), jnp.float32),
        compiler_params=pltpu.CompilerParams(use_tc_tiling_on_sc=False),
        scratch_types=[
            pltpu.VMEM((n_ch, _CH), jnp.int32),
            pltpu.VMEM((2, _CH, _D), jnp.float32),
            pltpu.SemaphoreType.DMA,
            pltpu.SemaphoreType.DMA,
            pltpu.SemaphoreType.DMA,
            pltpu.SemaphoreType.DMA,
        ],
    )
    def emb(idx_hbm, tbl_hbm, out_hbm, idx_v, rows_v, g0, g1, s0, s1):
        wid = lax.axis_index("s") * nc + lax.axis_index("c")
        base = wid * b_per_w
        pltpu.sync_copy(idx_hbm.at[wid], idx_v)

        gsem = (g0, g1)
        ssem = (s0, s1)

        def fire_gather(j, slot):
            return pltpu.async_copy(
                tbl_hbm.at[idx_v.at[j]], rows_v.at[slot], gsem[slot])

        def fire_store(j, slot):
            return pltpu.async_copy(
                rows_v.at[slot], out_hbm.at[pl.ds(base + j * _CH, _CH)],
                ssem[slot])

        gh = [None, None]
        sh = [None, None]
        gh[0] = fire_gather(0, 0)
        for j in range(n_ch):
            slot = j % 2
            other = 1 - slot
            if j + 1 < n_ch:
                if sh[other] is not None:
                    sh[other].wait()
                gh[other] = fire_gather(j + 1, other)
            gh[slot].wait()
            sh[slot] = fire_store(j, slot)
        for h in sh:
            if h is not None:
                h.wait()

    return emb


def kernel(x, weight):
    b, cols = x.shape
    total = b * cols
    n_emb = weight.shape[0]
    info = plsc.get_sparse_core_info()
    nw = info.num_cores * info.num_subcores
    b_per_w = total // nw
    n_ch = b_per_w // _CH
    assert b_per_w * nw == total and n_ch * _CH == b_per_w

    wt = jnp.transpose(weight)
    tail = jnp.pad(lax.slice(wt, (0, n_emb - 64), (_D, n_emb)),
                   ((0, 0), (0, 64)))
    ta = _detrans_call(n_emb)(wt, tail)
    table_rm = ta.reshape(n_emb, _D)

    idx = x.reshape(nw, n_ch, _CH).astype(jnp.int32)
    out = _emb_call(total, n_ch, b_per_w, n_emb)(idx, table_rm)
    return out.reshape(b, cols, _D)


# speedup vs baseline: 1.2052x; 1.2052x over previous
import functools

import jax
import jax.numpy as jnp
from jax import lax
from jax.experimental import pallas as pl
from jax.experimental.pallas import tpu as pltpu
from jax.experimental.pallas import tpu_sc as plsc

_D = 32
_CH = 1024
_LANES = 16


def _detrans_call(n_emb):
    mesh = plsc.VectorSubcoreMesh(core_axis_name="c", subcore_axis_name="s")
    info = plsc.get_sparse_core_info()
    nc = info.num_cores
    nw = nc * info.num_subcores
    n_slab = (n_emb + 127) // 128
    n_rows_out = (n_emb * _D) // 128
    n_full = n_slab - 1
    n_j = (n_full + nw - 1) // nw

    @functools.partial(
        pl.kernel,
        mesh=mesh,
        out_type=jax.ShapeDtypeStruct((n_rows_out, 128), jnp.float32),
        compiler_params=pltpu.CompilerParams(
            use_tc_tiling_on_sc=True, needs_layout_passes=False),
        scratch_types=[
            pltpu.VMEM((2, _D, 128), jnp.float32),
            pltpu.VMEM((2, _D, 128), jnp.float32),
            pltpu.SemaphoreType.DMA,
            pltpu.SemaphoreType.DMA,
            pltpu.SemaphoreType.DMA,
            pltpu.SemaphoreType.DMA,
        ],
    )
    def detrans(wt_hbm, tail_hbm, ta_hbm, vin, vout, i0, i1, o0, o1):
        wid = lax.axis_index("s") * nc + lax.axis_index("c")
        isem = (i0, i1)
        osem = (o0, o1)
        iota = lax.iota(jnp.int32, _LANES)
        row_half = (iota, iota + _LANES)

        def transpose_slab(slot, n_t):
            for t in range(n_t):
                for g in range(8):
                    rows = row_half[g % 2]
                    cols = jnp.full((_LANES,), 4 * t + g // 2, jnp.int32)
                    val = plsc.load_gather(vin.at[slot], [rows, cols])
                    vout[slot, t, pl.ds(16 * g, 16)] = val

        def fire_in(j, slot):
            sl = wid + nw * j
            pltpu.async_copy(
                wt_hbm.at[:, pl.ds(sl * 128, 128)], vin.at[slot], isem[slot])

        def wait_in(slot):
            pltpu.make_async_copy(
                wt_hbm.at[:, pl.ds(0, 128)], vin.at[slot], isem[slot]).wait()

        def fire_out(j, slot):
            sl = wid + nw * j
            pltpu.async_copy(
                vout.at[slot], ta_hbm.at[pl.ds(sl * _D, _D)], osem[slot])

        def wait_out(slot):
            pltpu.make_async_copy(
                vout.at[slot], ta_hbm.at[pl.ds(0, _D)], osem[slot]).wait()

        def valid(j):
            return wid + nw * j < n_full

        def half(j, slot):
            @pl.when(valid(j + 1))
            def _prefetch():
                fire_in(j + 1, 1 - slot)

            @pl.when(valid(j))
            def _work():
                wait_in(slot)

                @pl.when(j >= 2)
                def _drain_prev():
                    wait_out(slot)

                transpose_slab(slot, _D)
                fire_out(j, slot)

        fire_in(0, 0)

        def step(p, carry):
            half(2 * p, 0)
            half(2 * p + 1, 1)
            return carry

        lax.fori_loop(0, n_j // 2, step, 0)
        if n_j % 2:
            half(n_j - 1, 0)
        wait_out(0)
        wait_out(1)

        @pl.when(wid == 0)
        def _tail():
            pltpu.sync_copy(tail_hbm, vin.at[0])
            transpose_slab(0, _D // 2)
            pltpu.sync_copy(vout.at[0, pl.ds(0, _D // 2)],
                            ta_hbm.at[pl.ds((n_slab - 1) * _D, _D // 2)])

    return detrans


def _emb_call(total, n_ch, b_per_w, n_emb):
    mesh = plsc.VectorSubcoreMesh(core_axis_name="c", subcore_axis_name="s")
    info = plsc.get_sparse_core_info()
    nc = info.num_cores

    @functools.partial(
        pl.kernel,
        mesh=mesh,
        out_type=jax.ShapeDtypeStruct((total, _D), jnp.float32),
        compiler_params=pltpu.CompilerParams(use_tc_tiling_on_sc=False),
        scratch_types=[
            pltpu.VMEM((n_ch, _CH), jnp.int32),
            pltpu.VMEM((2, _CH, _D), jnp.float32),
            pltpu.SemaphoreType.DMA,
            pltpu.SemaphoreType.DMA,
            pltpu.SemaphoreType.DMA,
            pltpu.SemaphoreType.DMA,
        ],
    )
    def emb(idx_hbm, tbl_hbm, out_hbm, idx_v, rows_v, g0, g1, s0, s1):
        wid = lax.axis_index("s") * nc + lax.axis_index("c")
        base = wid * b_per_w
        pltpu.sync_copy(idx_hbm.at[wid], idx_v)

        gsem = (g0, g1)
        ssem = (s0, s1)

        def fire_gather(j, slot):
            return pltpu.async_copy(
                tbl_hbm.at[idx_v.at[j]], rows_v.at[slot], gsem[slot])

        def fire_store(j, slot):
            return pltpu.async_copy(
                rows_v.at[slot], out_hbm.at[pl.ds(base + j * _CH, _CH)],
                ssem[slot])

        gh = [None, None]
        sh = [None, None]
        gh[0] = fire_gather(0, 0)
        for j in range(n_ch):
            slot = j % 2
            other = 1 - slot
            if j + 1 < n_ch:
                if sh[other] is not None:
                    sh[other].wait()
                gh[other] = fire_gather(j + 1, other)
            gh[slot].wait()
            sh[slot] = fire_store(j, slot)
        for h in sh:
            if h is not None:
                h.wait()

    return emb


def kernel(x, weight):
    b, cols = x.shape
    total = b * cols
    n_emb = weight.shape[0]
    info = plsc.get_sparse_core_info()
    nw = info.num_cores * info.num_subcores
    b_per_w = total // nw
    n_ch = b_per_w // _CH
    assert b_per_w * nw == total and n_ch * _CH == b_per_w

    wt = jnp.transpose(weight)
    tail = jnp.pad(lax.slice(wt, (0, n_emb - 64), (_D, n_emb)),
                   ((0, 0), (0, 64)))
    ta = _detrans_call(n_emb)(wt, tail)
    table_rm = ta.reshape(n_emb, _D)

    idx = x.reshape(nw, n_ch, _CH).astype(jnp.int32)
    out = _emb_call(total, n_ch, b_per_w, n_emb)(idx, table_rm)
    return out.reshape(b, cols, _D)


# speedup vs baseline: 1.7283x; 1.4341x over previous
import functools

import jax
import jax.numpy as jnp
from jax import lax
from jax.experimental import pallas as pl
from jax.experimental.pallas import tpu as pltpu
from jax.experimental.pallas import tpu_sc as plsc

_D = 32
_CH = 1024


def _emb_call(total, n_ch, b_per_w):
    mesh = plsc.VectorSubcoreMesh(core_axis_name="c", subcore_axis_name="s")
    info = plsc.get_sparse_core_info()
    nc = info.num_cores

    @functools.partial(
        pl.kernel,
        mesh=mesh,
        out_type=jax.ShapeDtypeStruct((total, _D), jnp.float32),
        compiler_params=pltpu.CompilerParams(use_tc_tiling_on_sc=False),
        scratch_types=[
            pltpu.VMEM((n_ch, _CH), jnp.int32),
            pltpu.VMEM((2, _CH, _D), jnp.float32),
            pltpu.SemaphoreType.DMA,
            pltpu.SemaphoreType.DMA,
            pltpu.SemaphoreType.DMA,
            pltpu.SemaphoreType.DMA,
        ],
    )
    def emb(idx_hbm, tbl_hbm, out_hbm, idx_v, rows_v, g0, g1, s0, s1):
        wid = lax.axis_index("s") * nc + lax.axis_index("c")
        base = wid * b_per_w
        pltpu.sync_copy(idx_hbm.at[wid], idx_v)

        gsem = (g0, g1)
        ssem = (s0, s1)

        def fire_gather(j, slot):
            return pltpu.async_copy(
                tbl_hbm.at[idx_v.at[j]], rows_v.at[slot], gsem[slot])

        def fire_store(j, slot):
            return pltpu.async_copy(
                rows_v.at[slot], out_hbm.at[pl.ds(base + j * _CH, _CH)],
                ssem[slot])

        gh = [None, None]
        sh = [None, None]
        gh[0] = fire_gather(0, 0)
        for j in range(n_ch):
            slot = j % 2
            other = 1 - slot
            if j + 1 < n_ch:
                if sh[other] is not None:
                    sh[other].wait()
                gh[other] = fire_gather(j + 1, other)
            gh[slot].wait()
            sh[slot] = fire_store(j, slot)
        for h in sh:
            if h is not None:
                h.wait()

    return emb


def kernel(x, weight):
    b, cols = x.shape
    total = b * cols
    info = plsc.get_sparse_core_info()
    nw = info.num_cores * info.num_subcores
    b_per_w = total // nw
    n_ch = b_per_w // _CH
    assert b_per_w * nw == total and n_ch * _CH == b_per_w

    idx = x.reshape(nw, n_ch, _CH).astype(jnp.int32)
    out = _emb_call(total, n_ch, b_per_w)(idx, weight)
    return out.reshape(b, cols, _D)
